# baseline (speedup 1.0000x reference)
import functools

import jax
import jax.numpy as jnp
from jax import lax
from jax.experimental import pallas as pl
from jax.experimental.pallas import tpu as pltpu
from jax.experimental.pallas import tpu_sc as plsc

SIGMA = 0.4
N_IN = 40962
N_OUT = 163842
C = 128
K = 7
NW = 32
G = 48
CPW = 108
RPW = G * CPW
STRIDE = 5121
LAST_START = N_OUT - RPW
GK = G * K
IDX_MINOR = 112
NSEG = GK // IDX_MINOR


def _sc_body(x_hbm, idx_hbm, dm_hbm, mk_hbm, out_hbm,
             idx_v, dm_v, mk_v, gath_v, outb_v,
             sem_in0, sem_in1, sem_g0, sem_g1, sem_o0, sem_o1):
    cid = lax.axis_index("c")
    sid = lax.axis_index("s")
    wid = sid * 2 + cid
    start = jnp.minimum(wid * STRIDE, LAST_START)
    sem_in = [sem_in0, sem_in1]
    sem_g = [sem_g0, sem_g1]
    sem_o = [sem_o0, sem_o1]
    c1 = -1.0 / (2.0 * SIGMA * SIGMA)
    iota7 = lax.iota(jnp.int32, 16) * K

    def fire_in(c, b):
        pltpu.async_copy(idx_hbm.at[wid, c], idx_v.at[b], sem_in[b])
        pltpu.async_copy(dm_hbm.at[wid, c], dm_v.at[b], sem_in[b])
        pltpu.async_copy(mk_hbm.at[wid, c], mk_v.at[b], sem_in[b])

    def wait_in(c, b):
        pltpu.make_async_copy(idx_hbm.at[wid, c], idx_v.at[b], sem_in[b]).wait()
        pltpu.make_async_copy(dm_hbm.at[wid, c], dm_v.at[b], sem_in[b]).wait()
        pltpu.make_async_copy(mk_hbm.at[wid, c], mk_v.at[b], sem_in[b]).wait()

    def fire_gath(b):
        for j in range(NSEG):
            pltpu.async_copy(x_hbm.at[idx_v.at[b, j]],
                             gath_v.at[b, pl.ds(j * IDX_MINOR, IDX_MINOR)],
                             sem_g[b])

    def wait_gath(b):
        for j in range(NSEG):
            pltpu.make_async_copy(
                x_hbm.at[idx_v.at[b, j]],
                gath_v.at[b, pl.ds(j * IDX_MINOR, IDX_MINOR)],
                sem_g[b]).wait()

    def out_desc(c, b):
        return pltpu.make_async_copy(
            outb_v.at[b], out_hbm.at[pl.ds(start + c * G, G)], sem_o[b])

    def compute(c, b):
        def group_body(j, acc_c):
            g0 = j * 16
            off = g0 * K + iota7
            wks = []
            for k in range(K):
                d = plsc.load_gather(dm_v.at[b], [off + k])
                m = plsc.load_gather(mk_v.at[b], [off + k])
                wks.append(jnp.exp(d * d * c1) * m)
            wsum = wks[0]
            for k in range(1, K):
                wsum = wsum + wks[k]
            inv = 1.0 / jnp.maximum(wsum, 1e-8)
            swks = [wk * inv for wk in wks]
            for r in range(16):
                base = (g0 + r) * K
                ws = [swks[k][r] for k in range(K)]
                for cc in range(C // 16):
                    csl = pl.ds(cc * 16, 16)
                    acc = ws[0] * gath_v[b, base, csl]
                    for k in range(1, K):
                        acc = acc + ws[k] * gath_v[b, base + k, csl]
                    outb_v[b, g0 + r, csl] = acc
            return acc_c

        lax.fori_loop(0, G // 16, group_body, 0)

    fire_in(0, 0)
    wait_in(0, 0)
    fire_gath(0)
    fire_in(1, 1)

    def pair_body(it, carry):
        c0 = it * 2
        for b in range(2):
            c = c0 + b
            nb = 1 - b
            @pl.when(c + 1 < CPW)
            def _():
                wait_in(c + 1, nb)
                fire_gath(nb)

            wait_gath(b)

            @pl.when(c >= 2)
            def _():
                out_desc(c - 2, b).wait()

            compute(c, b)
            out_desc(c, b).start()

            @pl.when(c + 2 < CPW)
            def _():
                fire_in(c + 2, b)
        return carry

    lax.fori_loop(0, CPW // 2, pair_body, 0)
    out_desc(CPW - 2, 0).wait()
    out_desc(CPW - 1, 1).wait()


def kernel(x, cand_idx, cand_mask, delta):
    x2 = x.reshape(N_IN, C)
    idx32 = cand_idx.astype(jnp.int32)
    starts = [min(w * STRIDE, LAST_START) for w in range(NW)]

    def pack(a):
        return jnp.stack(
            [lax.slice(a, (s, 0), (s + RPW, K)) for s in starts])

    idx_p = pack(idx32).reshape(NW, CPW, NSEG, IDX_MINOR)
    dm_p = pack(delta).reshape(NW, CPW, GK)
    mk_p = pack(cand_mask).reshape(NW, CPW, GK)

    sc_fn = functools.partial(
        pl.kernel,
        mesh=plsc.VectorSubcoreMesh(core_axis_name="c", subcore_axis_name="s"),
        out_type=jax.ShapeDtypeStruct((N_OUT, C), jnp.float32),
        scratch_types=[
            pltpu.VMEM((2, NSEG, IDX_MINOR), jnp.int32),
            pltpu.VMEM((2, GK), jnp.float32),
            pltpu.VMEM((2, GK), jnp.float32),
            pltpu.VMEM((2, GK, C), jnp.float32),
            pltpu.VMEM((2, G, C), jnp.float32),
            pltpu.SemaphoreType.DMA,
            pltpu.SemaphoreType.DMA,
            pltpu.SemaphoreType.DMA,
            pltpu.SemaphoreType.DMA,
            pltpu.SemaphoreType.DMA,
            pltpu.SemaphoreType.DMA,
        ],
        compiler_params=pltpu.CompilerParams(
            use_tc_tiling_on_sc=False, needs_layout_passes=False),
    )(_sc_body)
    out = sc_fn(x2, idx_p, dm_p, mk_p)
    return out.reshape(1, N_OUT, C)


# speedup vs baseline: 1.9530x; 1.9530x over previous
import functools

import jax
import jax.numpy as jnp
from jax import lax
from jax.experimental import pallas as pl
from jax.experimental.pallas import tpu as pltpu
from jax.experimental.pallas import tpu_sc as plsc

SIGMA = 0.4
N_IN = 40962
N_OUT = 163842
C = 128
K = 7
NW = 32
G = 48
CPW = 108
RPW = G * CPW
STRIDE = 5121
LAST_START = N_OUT - RPW
GK = G * K
IDX_MINOR = 112
NSEG = GK // IDX_MINOR


def _sc_body(x_hbm, idx_hbm, dm_hbm, mk_hbm, out_hbm,
             idx_v, dm_v, mk_v, gath_v, outb_v,
             sem_in0, sem_in1, sem_g0, sem_g1, sem_o0, sem_o1):
    cid = lax.axis_index("c")
    sid = lax.axis_index("s")
    wid = sid * 2 + cid
    start = jnp.minimum(wid * STRIDE, LAST_START)
    sem_in = [sem_in0, sem_in1]
    sem_g = [sem_g0, sem_g1]
    sem_o = [sem_o0, sem_o1]
    c1 = -1.0 / (2.0 * SIGMA * SIGMA)
    iota7 = lax.iota(jnp.int32, 16) * K

    def fire_in(c, b):
        pltpu.async_copy(idx_hbm.at[wid, c], idx_v.at[b], sem_in[b])
        pltpu.async_copy(dm_hbm.at[wid, c], dm_v.at[b], sem_in[b])
        pltpu.async_copy(mk_hbm.at[wid, c], mk_v.at[b], sem_in[b])

    def wait_in(c, b):
        pltpu.make_async_copy(idx_hbm.at[wid, c], idx_v.at[b], sem_in[b]).wait()
        pltpu.make_async_copy(dm_hbm.at[wid, c], dm_v.at[b], sem_in[b]).wait()
        pltpu.make_async_copy(mk_hbm.at[wid, c], mk_v.at[b], sem_in[b]).wait()

    def fire_gath(b):
        for j in range(NSEG):
            pltpu.async_copy(x_hbm.at[idx_v.at[b, j]],
                             gath_v.at[b, pl.ds(j * IDX_MINOR, IDX_MINOR)],
                             sem_g[b])

    def wait_gath(b):
        for j in range(NSEG):
            pltpu.make_async_copy(
                x_hbm.at[idx_v.at[b, j]],
                gath_v.at[b, pl.ds(j * IDX_MINOR, IDX_MINOR)],
                sem_g[b]).wait()

    def out_desc(c, b):
        return pltpu.make_async_copy(
            outb_v.at[b], out_hbm.at[pl.ds(start + c * G, G)], sem_o[b])

    def compute(c, b):
        @plsc.parallel_loop(0, G // 16)
        def group_body(j):
            g0 = j * 16
            off = g0 * K + iota7
            wks = []
            for k in range(K):
                d = plsc.load_gather(dm_v.at[b], [off + k])
                m = plsc.load_gather(mk_v.at[b], [off + k])
                wks.append(jnp.exp(d * d * c1) * m)
            wsum = wks[0]
            for k in range(1, K):
                wsum = wsum + wks[k]
            inv = 1.0 / jnp.maximum(wsum, 1e-8)
            swks = [wk * inv for wk in wks]
            for r in range(16):
                base = (g0 + r) * K
                ws = [swks[k][r] for k in range(K)]
                accs = []
                for cc in range(C // 16):
                    csl = pl.ds(cc * 16, 16)
                    p = [ws[k] * gath_v[b, base + k, csl] for k in range(K)]
                    s01 = p[0] + p[1]
                    s23 = p[2] + p[3]
                    s45 = p[4] + p[5]
                    accs.append((s01 + s23) + (s45 + p[6]))
                for cc in range(C // 16):
                    outb_v[b, g0 + r, pl.ds(cc * 16, 16)] = accs[cc]

    fire_in(0, 0)
    wait_in(0, 0)
    fire_gath(0)
    fire_in(1, 1)

    def pair_body(it, carry):
        c0 = it * 2
        for b in range(2):
            c = c0 + b
            nb = 1 - b
            @pl.when(c + 1 < CPW)
            def _():
                wait_in(c + 1, nb)
                fire_gath(nb)

            wait_gath(b)

            @pl.when(c >= 2)
            def _():
                out_desc(c - 2, b).wait()

            compute(c, b)
            out_desc(c, b).start()

            @pl.when(c + 2 < CPW)
            def _():
                fire_in(c + 2, b)
        return carry

    lax.fori_loop(0, CPW // 2, pair_body, 0)
    out_desc(CPW - 2, 0).wait()
    out_desc(CPW - 1, 1).wait()


def kernel(x, cand_idx, cand_mask, delta):
    x2 = x.reshape(N_IN, C)
    idx32 = cand_idx.astype(jnp.int32)
    starts = [min(w * STRIDE, LAST_START) for w in range(NW)]

    def pack(a):
        return jnp.stack(
            [lax.slice(a, (s, 0), (s + RPW, K)) for s in starts])

    idx_p = pack(idx32).reshape(NW, CPW, NSEG, IDX_MINOR)
    dm_p = pack(delta).reshape(NW, CPW, GK)
    mk_p = pack(cand_mask).reshape(NW, CPW, GK)

    sc_fn = functools.partial(
        pl.kernel,
        mesh=plsc.VectorSubcoreMesh(core_axis_name="c", subcore_axis_name="s"),
        out_type=jax.ShapeDtypeStruct((N_OUT, C), jnp.float32),
        scratch_types=[
            pltpu.VMEM((2, NSEG, IDX_MINOR), jnp.int32),
            pltpu.VMEM((2, GK), jnp.float32),
            pltpu.VMEM((2, GK), jnp.float32),
            pltpu.VMEM((2, GK, C), jnp.float32),
            pltpu.VMEM((2, G, C), jnp.float32),
            pltpu.SemaphoreType.DMA,
            pltpu.SemaphoreType.DMA,
            pltpu.SemaphoreType.DMA,
            pltpu.SemaphoreType.DMA,
            pltpu.SemaphoreType.DMA,
            pltpu.SemaphoreType.DMA,
        ],
        compiler_params=pltpu.CompilerParams(
            use_tc_tiling_on_sc=False, needs_layout_passes=False),
    )(_sc_body)
    out = sc_fn(x2, idx_p, dm_p, mk_p)
    return out.reshape(1, N_OUT, C)
